# Initial kernel scaffold; baseline (speedup 1.0000x reference)
import jax
import jax.numpy as jnp
from jax.experimental import pallas as pl


def kernel(fine_features, coarse_features, atom_to_coarse, global_features, W_f2c, b_f2c, g1, be1, W_c2f, b_c2f, g2, be2, W_gate, b_gate, W_gi, b_gi, g3, be3):
    raise NotImplementedError("write your pallas kernel here")


# speedup vs baseline: 6.0953x; 6.0953x over previous
import functools

import jax
import jax.numpy as jnp
from jax.experimental import pallas as pl
from jax.experimental.pallas import tpu as pltpu


def _ln(x, g, b):
    m = jnp.mean(x, axis=-1, keepdims=True)
    v = jnp.mean((x - m) ** 2, axis=-1, keepdims=True)
    return (x - m) / jnp.sqrt(v + 1e-5) * g + b


def _fused_kernel(fine_ref, glob_ref, idx_ref, coarse_ref,
                  wf2c_ref, bf2c_ref, g1_ref, be1_ref,
                  wc2f_ref, bc2f_ref, g2_ref, be2_ref,
                  wga_ref, wgb_ref, bg_ref,
                  wgia_ref, wgib_ref, bgi_ref, g3_ref, be3_ref,
                  out_fine_ref, out_coarse_ref,
                  sums_ref, counts_ref, gsum_ref,
                  *, num_tiles, n_total, nc):
    t = pl.program_id(1)

    f = fine_ref[0]
    gl = glob_ref[0]
    cb = coarse_ref[0]
    ids = idx_ref[0]

    iota_c = jax.lax.broadcasted_iota(jnp.int32, (nc, ids.shape[-1]), 0)
    m_t = (iota_c == ids).astype(jnp.float32)

    dot = functools.partial(jnp.dot, preferred_element_type=jnp.float32)

    ffc0 = jax.lax.dot_general(
        m_t, cb, (((0,), (0,)), ((), ())),
        preferred_element_type=jnp.float32)
    ffc = jax.nn.relu(_ln(dot(ffc0, wc2f_ref[...]) + bc2f_ref[...],
                          g2_ref[...], be2_ref[...]))

    z = dot(f, wga_ref[...]) + dot(ffc, wgb_ref[...]) + bg_ref[...]
    gate = jax.nn.sigmoid(z)
    fu = gate * f + (1.0 - gate) * ffc

    z2 = dot(fu, wgia_ref[...]) + dot(gl, wgib_ref[...]) + bgi_ref[...]
    fwg = jax.nn.relu(_ln(z2, g3_ref[...], be3_ref[...]))
    out_fine_ref[0] = fu + 0.1 * fwg

    part_sums = dot(m_t, f)
    part_counts = jnp.sum(m_t, axis=1, keepdims=True)
    part_gsum = jnp.sum(gl, axis=0, keepdims=True)

    @pl.when(t == 0)
    def _():
        sums_ref[...] = part_sums
        counts_ref[...] = part_counts
        gsum_ref[...] = part_gsum

    @pl.when(t > 0)
    def _():
        sums_ref[...] += part_sums
        counts_ref[...] += part_counts
        gsum_ref[...] += part_gsum

    @pl.when(t == num_tiles - 1)
    def _():
        cnt = jnp.maximum(counts_ref[...], 1.0)
        cff0 = sums_ref[...] / cnt
        cff = jax.nn.relu(_ln(dot(cff0, wf2c_ref[...]) + bf2c_ref[...],
                              g1_ref[...], be1_ref[...]))
        zc = dot(cb, wga_ref[...]) + dot(cff, wgb_ref[...]) + bg_ref[...]
        cgate = jax.nn.sigmoid(zc)
        cu = cgate * cb + (1.0 - cgate) * cff
        gm = gsum_ref[...] * (1.0 / n_total)
        zc2 = dot(cu, wgia_ref[...]) + dot(gm, wgib_ref[...]) + bgi_ref[...]
        cwg = jax.nn.relu(_ln(zc2, g3_ref[...], be3_ref[...]))
        out_coarse_ref[0] = cu + 0.1 * cwg


def kernel(fine_features, coarse_features, atom_to_coarse, global_features,
           W_f2c, b_f2c, g1, be1, W_c2f, b_c2f, g2, be2,
           W_gate, b_gate, W_gi, b_gi, g3, be3):
    B, N, H = fine_features.shape
    NC = coarse_features.shape[1]
    TN = min(N, 512)
    T = N // TN

    idx3 = atom_to_coarse.reshape(B * T, 1, TN)
    row = lambda v: v.reshape(1, H)

    grid = (B, T)
    tile_spec = pl.BlockSpec((1, TN, H), lambda b, t: (b, t, 0))
    coarse_spec = pl.BlockSpec((1, NC, H), lambda b, t: (b, 0, 0))
    w_spec = pl.BlockSpec((H, H), lambda b, t: (0, 0))
    v_spec = pl.BlockSpec((1, H), lambda b, t: (0, 0))

    out_fine, out_coarse = pl.pallas_call(
        functools.partial(_fused_kernel, num_tiles=T, n_total=N, nc=NC),
        grid=grid,
        in_specs=[
            tile_spec,
            tile_spec,
            pl.BlockSpec((1, 1, TN), lambda b, t: (b * T + t, 0, 0)),
            coarse_spec,
            w_spec, v_spec, v_spec, v_spec,
            w_spec, v_spec, v_spec, v_spec,
            w_spec, w_spec, v_spec,
            w_spec, w_spec, v_spec, v_spec, v_spec,
        ],
        out_specs=[tile_spec, coarse_spec],
        out_shape=[
            jax.ShapeDtypeStruct((B, N, H), jnp.float32),
            jax.ShapeDtypeStruct((B, NC, H), jnp.float32),
        ],
        scratch_shapes=[
            pltpu.VMEM((NC, H), jnp.float32),
            pltpu.VMEM((NC, 1), jnp.float32),
            pltpu.VMEM((1, H), jnp.float32),
        ],
    )(fine_features, global_features, idx3, coarse_features,
      W_f2c, row(b_f2c), row(g1), row(be1),
      W_c2f, row(b_c2f), row(g2), row(be2),
      W_gate[:H], W_gate[H:], row(b_gate),
      W_gi[:H], W_gi[H:], row(b_gi), row(g3), row(be3))

    return (out_fine, out_coarse)


# speedup vs baseline: 6.1404x; 1.0074x over previous
import functools

import jax
import jax.numpy as jnp
from jax.experimental import pallas as pl
from jax.experimental.pallas import tpu as pltpu


def _ln(x, g, b):
    m = jnp.mean(x, axis=-1, keepdims=True)
    v = jnp.mean((x - m) ** 2, axis=-1, keepdims=True)
    return (x - m) / jnp.sqrt(v + 1e-5) * g + b


def _fused_kernel(fine_ref, glob_ref, idx_ref, coarse_ref,
                  wf2c_ref, bf2c_ref, g1_ref, be1_ref,
                  wc2f_ref, bc2f_ref, g2_ref, be2_ref,
                  wga_ref, wgb_ref, bg_ref,
                  wgia_ref, wgib_ref, bgi_ref, g3_ref, be3_ref,
                  out_fine_ref, out_coarse_ref,
                  sums_ref, counts_ref, gsum_ref, tbl_ref,
                  *, num_tiles, n_total, nc):
    t = pl.program_id(1)

    f = fine_ref[0]
    gl = glob_ref[0]
    cb = coarse_ref[0]
    ids = idx_ref[0]

    iota_c = jax.lax.broadcasted_iota(jnp.int32, (nc, ids.shape[-1]), 0)
    m_t = (iota_c == ids).astype(jnp.float32)

    dot = functools.partial(jnp.dot, preferred_element_type=jnp.float32)

    @pl.when(t == 0)
    def _():
        tbl_ref[...] = jax.nn.relu(
            _ln(dot(cb, wc2f_ref[...]) + bc2f_ref[...],
                g2_ref[...], be2_ref[...]))

    ffc = jax.lax.dot_general(
        m_t, tbl_ref[...], (((0,), (0,)), ((), ())),
        preferred_element_type=jnp.float32)

    z = dot(f, wga_ref[...]) + dot(ffc, wgb_ref[...]) + bg_ref[...]
    gate = jax.nn.sigmoid(z)
    fu = gate * f + (1.0 - gate) * ffc

    z2 = dot(fu, wgia_ref[...]) + dot(gl, wgib_ref[...]) + bgi_ref[...]
    fwg = jax.nn.relu(_ln(z2, g3_ref[...], be3_ref[...]))
    out_fine_ref[0] = fu + 0.1 * fwg

    part_sums = dot(m_t, f)
    part_counts = jnp.sum(m_t, axis=1, keepdims=True)
    part_gsum = jnp.sum(gl, axis=0, keepdims=True)

    @pl.when(t == 0)
    def _():
        sums_ref[...] = part_sums
        counts_ref[...] = part_counts
        gsum_ref[...] = part_gsum

    @pl.when(t > 0)
    def _():
        sums_ref[...] += part_sums
        counts_ref[...] += part_counts
        gsum_ref[...] += part_gsum

    @pl.when(t == num_tiles - 1)
    def _():
        cnt = jnp.maximum(counts_ref[...], 1.0)
        cff0 = sums_ref[...] / cnt
        cff = jax.nn.relu(_ln(dot(cff0, wf2c_ref[...]) + bf2c_ref[...],
                              g1_ref[...], be1_ref[...]))
        zc = dot(cb, wga_ref[...]) + dot(cff, wgb_ref[...]) + bg_ref[...]
        cgate = jax.nn.sigmoid(zc)
        cu = cgate * cb + (1.0 - cgate) * cff
        gm = gsum_ref[...] * (1.0 / n_total)
        zc2 = dot(cu, wgia_ref[...]) + dot(gm, wgib_ref[...]) + bgi_ref[...]
        cwg = jax.nn.relu(_ln(zc2, g3_ref[...], be3_ref[...]))
        out_coarse_ref[0] = cu + 0.1 * cwg


def kernel(fine_features, coarse_features, atom_to_coarse, global_features,
           W_f2c, b_f2c, g1, be1, W_c2f, b_c2f, g2, be2,
           W_gate, b_gate, W_gi, b_gi, g3, be3):
    B, N, H = fine_features.shape
    NC = coarse_features.shape[1]
    TN = min(N, 512)
    T = N // TN

    idx3 = atom_to_coarse.reshape(B * T, 1, TN)
    row = lambda v: v.reshape(1, H)

    grid = (B, T)
    tile_spec = pl.BlockSpec((1, TN, H), lambda b, t: (b, t, 0))
    coarse_spec = pl.BlockSpec((1, NC, H), lambda b, t: (b, 0, 0))
    w_spec = pl.BlockSpec((H, H), lambda b, t: (0, 0))
    v_spec = pl.BlockSpec((1, H), lambda b, t: (0, 0))

    out_fine, out_coarse = pl.pallas_call(
        functools.partial(_fused_kernel, num_tiles=T, n_total=N, nc=NC),
        grid=grid,
        in_specs=[
            tile_spec,
            tile_spec,
            pl.BlockSpec((1, 1, TN), lambda b, t: (b * T + t, 0, 0)),
            coarse_spec,
            w_spec, v_spec, v_spec, v_spec,
            w_spec, v_spec, v_spec, v_spec,
            w_spec, w_spec, v_spec,
            w_spec, w_spec, v_spec, v_spec, v_spec,
        ],
        out_specs=[tile_spec, coarse_spec],
        out_shape=[
            jax.ShapeDtypeStruct((B, N, H), jnp.float32),
            jax.ShapeDtypeStruct((B, NC, H), jnp.float32),
        ],
        scratch_shapes=[
            pltpu.VMEM((NC, H), jnp.float32),
            pltpu.VMEM((NC, 1), jnp.float32),
            pltpu.VMEM((1, H), jnp.float32),
            pltpu.VMEM((NC, H), jnp.float32),
        ],
    )(fine_features, global_features, idx3, coarse_features,
      W_f2c, row(b_f2c), row(g1), row(be1),
      W_c2f, row(b_c2f), row(g2), row(be2),
      W_gate[:H], W_gate[H:], row(b_gate),
      W_gi[:H], W_gi[H:], row(b_gi), row(g3), row(be3))

    return (out_fine, out_coarse)
